# Initial kernel scaffold; baseline (speedup 1.0000x reference)
import functools

import jax
import jax.numpy as jnp
from jax import lax
from jax.experimental import pallas as pl
from jax.experimental.pallas import tpu as pltpu
from jax.experimental.pallas import tpu_sc as plsc

_LOG2 = 0.6931471805599453


def _ssp(x):
    return jnp.maximum(x, 0.0) + jnp.log(1.0 + jnp.exp(-jnp.abs(x))) - _LOG2


def _mlp_body(x_ref, w1_ref, b1_ref, w2_ref, b2_ref, q_ref):
    h = jnp.dot(x_ref[...], w1_ref[...], preferred_element_type=jnp.float32)
    h = _ssp(h + b1_ref[...])
    g = jnp.dot(h, w2_ref[...], preferred_element_type=jnp.float32)
    q_ref[...] = _ssp(g + b2_ref[...])


def _mlp(x2, W1, b1, W2, b2, BM=1000):
    M, Fa = x2.shape
    Fd = W2.shape[1]
    grid = M // BM
    return pl.pallas_call(
        _mlp_body,
        grid=(grid,),
        in_specs=[
            pl.BlockSpec((BM, Fa), lambda i: (i, 0)),
            pl.BlockSpec((Fa, Fa), lambda i: (0, 0)),
            pl.BlockSpec((1, Fa), lambda i: (0, 0)),
            pl.BlockSpec((Fa, Fd), lambda i: (0, 0)),
            pl.BlockSpec((1, Fd), lambda i: (0, 0)),
        ],
        out_specs=pl.BlockSpec((BM, Fd), lambda i: (i, 0)),
        out_shape=jax.ShapeDtypeStruct((M, Fd), jnp.float32),
    )(x2, W1, b1.reshape(1, Fa), W2, b2.reshape(1, Fd))


_T = 16
_NCH = 4


def _sc_body(A, AP, N, Fd, q_hbm, nb_hbm, vt_hbm, m_hbm, out_hbm,
             q_tab, nb_buf, v_buf, m_buf, o_buf):
    c = lax.axis_index("c")
    s = lax.axis_index("s")
    wid = s * 2 + c
    b = wid // _NCH
    a0 = (wid % _NCH) * (AP // _NCH)
    pltpu.sync_copy(q_hbm.at[b], q_tab)
    ng = Fd // 16

    def tile(t, carry):
        a = a0 + t * _T
        pltpu.sync_copy(nb_hbm.at[b, pl.ds(a, _T)], nb_buf)
        pltpu.sync_copy(vt_hbm.at[b, pl.ds(a, _T)], v_buf)
        pltpu.sync_copy(m_hbm.at[b, pl.ds(a, _T)], m_buf)

        def atom(i, carry2):
            for d in range(3):
                for h in range(N // 16):
                    v_buf[i, d, pl.ds(h * 16, 16)] = (
                        v_buf[i, d, pl.ds(h * 16, 16)]
                        * m_buf[i, pl.ds(h * 16, 16)])
            acc = [jnp.zeros((16,), jnp.float32) for _ in range(3 * ng)]
            for n in range(N):
                base = nb_buf[i, n] * Fd
                qs = [q_tab[pl.ds(base + g * 16, 16)] for g in range(ng)]
                for d in range(3):
                    sv = v_buf[i, d, n]
                    for g in range(ng):
                        acc[d * ng + g] = acc[d * ng + g] + qs[g] * sv
            for d in range(3):
                for g in range(ng):
                    o_buf[i, d, pl.ds(g * 16, 16)] = acc[d * ng + g]
            return carry2

        lax.fori_loop(0, _T, atom, 0)
        pltpu.sync_copy(o_buf, out_hbm.at[b, pl.ds(a, _T)])
        return carry

    lax.fori_loop(0, (AP // _NCH) // _T, tile, 0)


def _sc_reduce(q2, nb_p, vt_p, m_p, A, AP, N, Fd):
    B = q2.shape[0]
    mesh = plsc.VectorSubcoreMesh(core_axis_name="c", subcore_axis_name="s")
    body = functools.partial(_sc_body, A, AP, N, Fd)
    f = pl.kernel(
        body,
        out_type=jax.ShapeDtypeStruct((B, AP, 3, Fd), jnp.float32),
        mesh=mesh,
        scratch_types=[
            pltpu.VMEM((A * Fd,), jnp.float32),
            pltpu.VMEM((_T, N), jnp.int32),
            pltpu.VMEM((_T, 3, N), jnp.float32),
            pltpu.VMEM((_T, N), jnp.float32),
            pltpu.VMEM((_T, 3, Fd), jnp.float32),
        ],
    )
    return f(q2, nb_p, vt_p, m_p)


def kernel(x, r_ij, v_ij, neighbors, neighbor_mask, W1, b1, W2, b2):
    B, A, Fa = x.shape
    N = neighbors.shape[-1]
    Fd = W2.shape[1]
    AP = 1280

    q = _mlp(x.reshape(B * A, Fa), W1, b1, W2, b2)
    q2 = q.reshape(B, A * Fd)

    pad = ((0, 0), (0, AP - A), (0, 0))
    nb_p = jnp.pad(neighbors.astype(jnp.int32), pad)
    vt_p = jnp.pad(jnp.swapaxes(v_ij, 2, 3), pad + ((0, 0),))
    m_p = jnp.pad(neighbor_mask, pad)

    mu_t = _sc_reduce(q2, nb_p, vt_p, m_p, A, AP, N, Fd)
    return jnp.swapaxes(mu_t[:, :A], 2, 3)


# speedup vs baseline: 12.6493x; 12.6493x over previous
import functools

import jax
import jax.numpy as jnp
from jax import lax
from jax.experimental import pallas as pl
from jax.experimental.pallas import tpu as pltpu
from jax.experimental.pallas import tpu_sc as plsc

_LOG2 = 0.6931471805599453


def _ssp(x):
    return jnp.maximum(x, 0.0) + jnp.log(1.0 + jnp.exp(-jnp.abs(x))) - _LOG2


def _mlp_body(x_ref, w1_ref, b1_ref, w2_ref, b2_ref, q_ref):
    h = jnp.dot(x_ref[...], w1_ref[...], preferred_element_type=jnp.float32)
    h = _ssp(h + b1_ref[...])
    g = jnp.dot(h, w2_ref[...], preferred_element_type=jnp.float32)
    q_ref[...] = _ssp(g + b2_ref[...])


def _mlp(x2, W1, b1, W2, b2, BM=1000):
    M, Fa = x2.shape
    Fd = W2.shape[1]
    grid = M // BM
    return pl.pallas_call(
        _mlp_body,
        grid=(grid,),
        in_specs=[
            pl.BlockSpec((BM, Fa), lambda i: (i, 0)),
            pl.BlockSpec((Fa, Fa), lambda i: (0, 0)),
            pl.BlockSpec((1, Fa), lambda i: (0, 0)),
            pl.BlockSpec((Fa, Fd), lambda i: (0, 0)),
            pl.BlockSpec((1, Fd), lambda i: (0, 0)),
        ],
        out_specs=pl.BlockSpec((BM, Fd), lambda i: (i, 0)),
        out_shape=jax.ShapeDtypeStruct((M, Fd), jnp.float32),
    )(x2, W1, b1.reshape(1, Fa), W2, b2.reshape(1, Fd))


_T = 16
_NCH = 4


def _sc_body(A, AP, N, Fd, q_hbm, nb_hbm, vt_hbm, m_hbm, out_hbm,
             q_tab, nb_buf, v_buf, m_buf, o_buf):
    c = lax.axis_index("c")
    s = lax.axis_index("s")
    wid = s * 2 + c
    b = wid // _NCH
    a0 = (wid % _NCH) * (AP // _NCH)
    pltpu.sync_copy(q_hbm.at[b], q_tab)
    ng = Fd // 16

    def tile(t, carry):
        a = a0 + t * _T
        pltpu.sync_copy(nb_hbm.at[b, pl.ds(a, _T)], nb_buf)
        pltpu.sync_copy(vt_hbm.at[b, pl.ds(a, _T)], v_buf)
        pltpu.sync_copy(m_hbm.at[b, pl.ds(a, _T)], m_buf)

        def atom(i, carry2):
            nh = N // 16
            nbv = [nb_buf[i, pl.ds(h * 16, 16)] for h in range(nh)]
            mv = [m_buf[i, pl.ds(h * 16, 16)] for h in range(nh)]
            vm = [[v_buf[i, d, pl.ds(h * 16, 16)] * mv[h] for h in range(nh)]
                  for d in range(3)]
            acc = [jnp.zeros((16,), jnp.float32) for _ in range(3 * ng)]
            for n in range(N):
                h, l = divmod(n, 16)
                base = nbv[h][l] * Fd
                qs = [q_tab[pl.ds(base + g * 16, 16)] for g in range(ng)]
                for d in range(3):
                    sv = vm[d][h][l]
                    for g in range(ng):
                        acc[d * ng + g] = acc[d * ng + g] + qs[g] * sv
            for d in range(3):
                for g in range(ng):
                    o_buf[i, d, pl.ds(g * 16, 16)] = acc[d * ng + g]
            return carry2

        lax.fori_loop(0, _T, atom, 0)
        pltpu.sync_copy(o_buf, out_hbm.at[b, pl.ds(a, _T)])
        return carry

    lax.fori_loop(0, (AP // _NCH) // _T, tile, 0)


def _sc_reduce(q2, nb_p, vt_p, m_p, A, AP, N, Fd):
    B = q2.shape[0]
    mesh = plsc.VectorSubcoreMesh(core_axis_name="c", subcore_axis_name="s")
    body = functools.partial(_sc_body, A, AP, N, Fd)
    f = pl.kernel(
        body,
        out_type=jax.ShapeDtypeStruct((B, AP, 3, Fd), jnp.float32),
        mesh=mesh,
        scratch_types=[
            pltpu.VMEM((A * Fd,), jnp.float32),
            pltpu.VMEM((_T, N), jnp.int32),
            pltpu.VMEM((_T, 3, N), jnp.float32),
            pltpu.VMEM((_T, N), jnp.float32),
            pltpu.VMEM((_T, 3, Fd), jnp.float32),
        ],
    )
    return f(q2, nb_p, vt_p, m_p)


def kernel(x, r_ij, v_ij, neighbors, neighbor_mask, W1, b1, W2, b2):
    B, A, Fa = x.shape
    N = neighbors.shape[-1]
    Fd = W2.shape[1]
    AP = 1280

    q = _mlp(x.reshape(B * A, Fa), W1, b1, W2, b2)
    q2 = q.reshape(B, A * Fd)

    pad = ((0, 0), (0, AP - A), (0, 0))
    nb_p = jnp.pad(neighbors.astype(jnp.int32), pad)
    vt_p = jnp.pad(jnp.swapaxes(v_ij, 2, 3), pad + ((0, 0),))
    m_p = jnp.pad(neighbor_mask, pad)

    mu_t = _sc_reduce(q2, nb_p, vt_p, m_p, A, AP, N, Fd)
    return jnp.swapaxes(mu_t[:, :A], 2, 3)
